# baseline (speedup 1.0000x reference)
import jax
import jax.numpy as jnp
from jax import lax
from jax.experimental import pallas as pl
from jax.experimental.pallas import tpu as pltpu

_MC = 256
_ALPHA_C = 1.0


def _chamfer_body(x_ref, y_ref, mask_ref, conf_ref,
                  conf_out, pred_out, gt_out, rowacc):
    x = x_ref[0]
    y = y_ref[0]
    V2 = x.shape[0]
    V1 = y.shape[1]

    x2 = jnp.sum(x * x, axis=1, keepdims=True)
    y2 = jnp.sum(y * y, axis=0, keepdims=True)
    xb = x.astype(jnp.bfloat16)
    yb = y.astype(jnp.bfloat16)

    rowacc[...] = jnp.full((V2, _MC), jnp.inf, jnp.float32)

    for j in range(V1 // _MC):
        sl = slice(j * _MC, (j + 1) * _MC)
        pj = lax.dot_general(
            xb, yb[:, sl], (((1,), (0,)), ((), ())),
            preferred_element_type=jnp.float32)
        dj = jnp.maximum(x2 + y2[:, sl] - 2.0 * pj, 0.0)
        rowacc[...] = jnp.minimum(rowacc[...], dj)
        colmin = jnp.min(dj, axis=0, keepdims=True)
        gt_out[0, :, sl] = jnp.sqrt(colmin) * 100.0

    rowmin = jnp.min(rowacc[...], axis=1, keepdims=True)
    lp = jnp.sqrt(rowmin) * 100.0
    m = mask_ref[0]
    c = conf_ref[0]
    pred_out[0] = lp * m
    conf_out[0] = (lp * c - _ALPHA_C * jnp.log(c)) * m


def kernel(x_gt, x_pred, mask, confidence):
    B, V1, _ = x_gt.shape
    V2 = x_pred.shape[1]
    xp = x_pred * mask[..., None]
    y_t = jnp.transpose(x_gt, (0, 2, 1))
    mask3 = mask[..., None]
    conf3 = confidence[..., None]

    conf_o, pred_o, gt_o = pl.pallas_call(
        _chamfer_body,
        grid=(B,),
        in_specs=[
            pl.BlockSpec((1, V2, 3), lambda b: (b, 0, 0)),
            pl.BlockSpec((1, 3, V1), lambda b: (b, 0, 0)),
            pl.BlockSpec((1, V2, 1), lambda b: (b, 0, 0)),
            pl.BlockSpec((1, V2, 1), lambda b: (b, 0, 0)),
        ],
        out_specs=[
            pl.BlockSpec((1, V2, 1), lambda b: (b, 0, 0)),
            pl.BlockSpec((1, V2, 1), lambda b: (b, 0, 0)),
            pl.BlockSpec((1, 1, V1), lambda b: (b, 0, 0)),
        ],
        out_shape=[
            jax.ShapeDtypeStruct((B, V2, 1), jnp.float32),
            jax.ShapeDtypeStruct((B, V2, 1), jnp.float32),
            jax.ShapeDtypeStruct((B, 1, V1), jnp.float32),
        ],
        scratch_shapes=[pltpu.VMEM((V2, _MC), jnp.float32)],
    )(xp, y_t, mask3, conf3)

    return (conf_o.reshape(B, V2), pred_o.reshape(B, V2), gt_o.reshape(B, V1))


# speedup vs baseline: 1.4385x; 1.4385x over previous
import jax
import jax.numpy as jnp
from jax import lax
from jax.experimental import pallas as pl

_MC = 256
_ALPHA_C = 1.0


def _split3(v):
    h = v.astype(jnp.bfloat16)
    r = v - h.astype(jnp.float32)
    l = r.astype(jnp.bfloat16)
    l2 = (r - l.astype(jnp.float32)).astype(jnp.bfloat16)
    return h, l, l2


def _aug_pair(v, vb):
    V = v.shape[1]
    n2 = jnp.sum(v * v, axis=0, keepdims=True)
    nh, nl, nl2 = _split3(n2)
    ones = jnp.ones((3, V), jnp.bfloat16)
    zeros = jnp.zeros((7, V), jnp.bfloat16)
    lhs = jnp.concatenate([vb, nh, nl, nl2, ones, zeros], axis=0)
    rhs = jnp.concatenate([-2.0 * vb, ones, nh, nl, nl2, zeros], axis=0)
    return lhs, rhs


def _chamfer_body(x_ref, y_ref, mask_ref, conf_ref,
                  conf_out, pred_out, gt_out):
    m = mask_ref[0]
    x = x_ref[0] * m
    y = y_ref[0]
    V2 = x.shape[1]
    V1 = y.shape[1]

    x_lhs, x_rhs = _aug_pair(x, x.astype(jnp.bfloat16))
    y_lhs, y_rhs = _aug_pair(y, y.astype(jnp.bfloat16))

    dn = (((0,), (0,)), ((), ()))

    for j in range(V2 // _MC):
        sl = slice(j * _MC, (j + 1) * _MC)
        dj = lax.dot_general(y_lhs, x_rhs[:, sl], dn,
                             preferred_element_type=jnp.float32)
        cmin = jnp.maximum(jnp.min(dj, axis=0, keepdims=True), 0.0)
        lp = jnp.sqrt(cmin) * 100.0
        mj = m[:, sl]
        cj = conf_ref[0, :, sl]
        pred_out[0, :, sl] = lp * mj
        conf_out[0, :, sl] = (lp * cj - _ALPHA_C * jnp.log(cj)) * mj

    for j in range(V1 // _MC):
        sl = slice(j * _MC, (j + 1) * _MC)
        dj = lax.dot_general(x_lhs, y_rhs[:, sl], dn,
                             preferred_element_type=jnp.float32)
        cmin = jnp.maximum(jnp.min(dj, axis=0, keepdims=True), 0.0)
        gt_out[0, :, sl] = jnp.sqrt(cmin) * 100.0


def kernel(x_gt, x_pred, mask, confidence):
    B, V1, _ = x_gt.shape
    V2 = x_pred.shape[1]
    x_t = jnp.transpose(x_pred, (0, 2, 1))
    y_t = jnp.transpose(x_gt, (0, 2, 1))
    mask3 = mask.reshape(B, 1, V2)
    conf3 = confidence.reshape(B, 1, V2)

    conf_o, pred_o, gt_o = pl.pallas_call(
        _chamfer_body,
        grid=(B,),
        in_specs=[
            pl.BlockSpec((1, 3, V2), lambda b: (b, 0, 0)),
            pl.BlockSpec((1, 3, V1), lambda b: (b, 0, 0)),
            pl.BlockSpec((1, 1, V2), lambda b: (b, 0, 0)),
            pl.BlockSpec((1, 1, V2), lambda b: (b, 0, 0)),
        ],
        out_specs=[
            pl.BlockSpec((1, 1, V2), lambda b: (b, 0, 0)),
            pl.BlockSpec((1, 1, V2), lambda b: (b, 0, 0)),
            pl.BlockSpec((1, 1, V1), lambda b: (b, 0, 0)),
        ],
        out_shape=[
            jax.ShapeDtypeStruct((B, 1, V2), jnp.float32),
            jax.ShapeDtypeStruct((B, 1, V2), jnp.float32),
            jax.ShapeDtypeStruct((B, 1, V1), jnp.float32),
        ],
    )(x_t, y_t, mask3, conf3)

    return (conf_o.reshape(B, V2), pred_o.reshape(B, V2), gt_o.reshape(B, V1))
